# baseline (speedup 1.0000x reference)
import functools

import jax
import jax.numpy as jnp
from jax import lax
from jax.experimental import pallas as pl
from jax.experimental.pallas import tpu as pltpu
from jax.experimental.pallas import tpu_sc as plsc

N_NODES = 10000
D = 128
N_EDGES = 320000

N_SUBCORES = 16
CHUNK = 128
SUPER = 16
N_SUPERS = 10
CHUNKS_PER_TILE = SUPER * N_SUPERS
EDGES_PER_TILE = CHUNKS_PER_TILE * CHUNK
E_PAD = EDGES_PER_TILE * N_SUBCORES
ROWS_PER_TILE_ACC = 632
ACC_ROWS = ROWS_PER_TILE_ACC * N_SUBCORES
OUT_ROWS_MAIN = 632
OUT_ROWS_LAST = N_NODES - 15 * OUT_ROWS_MAIN

TC_BLOCK = 1000


def _pre_body(h_ref, c_ref, wf_ref, bf_ref, out_ref):
    z = jnp.dot(h_ref[...], wf_ref[...], preferred_element_type=jnp.float32)
    out_ref[...] = jax.nn.sigmoid(z + bf_ref[...]) * c_ref[...]


def _pre(h, c, Wf, bf2d):
    return pl.pallas_call(
        _pre_body,
        grid=(N_NODES // TC_BLOCK,),
        in_specs=[
            pl.BlockSpec((TC_BLOCK, D), lambda i: (i, 0)),
            pl.BlockSpec((TC_BLOCK, D), lambda i: (i, 0)),
            pl.BlockSpec((D, D), lambda i: (0, 0)),
            pl.BlockSpec((1, D), lambda i: (0, 0)),
        ],
        out_specs=pl.BlockSpec((TC_BLOCK, D), lambda i: (i, 0)),
        out_shape=jax.ShapeDtypeStruct((N_NODES, D), jnp.float32),
    )(h, c, Wf, bf2d)


def _sc_body(h_hbm, prod_hbm, src_hbm, dst_hbm, zero_hbm,
             hs_out, fc_out,
             si0, si1, di0, di1, buf0, buf1, acc, sem0, sem1, isem):
    cid = lax.axis_index("c")
    sid = lax.axis_index("s")

    pltpu.sync_copy(zero_hbm.at[pl.ds(sid * ROWS_PER_TILE_ACC, ROWS_PER_TILE_ACC)],
                    acc.at[pl.ds(sid * ROWS_PER_TILE_ACC, ROWS_PER_TILE_ACC)])
    plsc.subcore_barrier()

    def run(table, out):
        si = (si0, si1)
        di = (di0, di1)
        bufs = (buf0, buf1)
        sems = (sem0, sem1)

        def stage(s, par):
            pltpu.async_copy(src_hbm.at[sid, pl.ds(s * SUPER, SUPER)],
                             si[par], isem)
            pltpu.async_copy(dst_hbm.at[sid, pl.ds(s * SUPER, SUPER)],
                             di[par], isem)

        def wait_idx(par):
            pltpu.make_async_copy(src_hbm.at[sid, pl.ds(0, SUPER)],
                                  si[par], isem).wait()
            pltpu.make_async_copy(src_hbm.at[sid, pl.ds(0, SUPER)],
                                  di[par], isem).wait()

        def start(par, row, b):
            pltpu.async_copy(table.at[si[par].at[row]], bufs[b], sems[b])

        def wait(b):
            pltpu.make_async_copy(table.at[si[0].at[0]], bufs[b], sems[b]).wait()

        def scatter_add(par, row, b):
            pltpu.sync_copy(bufs[b], acc.at[di[par].at[row]], add=True)

        stage(0, 0)
        wait_idx(0)
        stage(1, 1)
        start(0, 0, 0)
        start(0, 1, 1)

        def pair(t, carry):
            for half in range(2):
                s = 2 * t + half
                par = half

                if half == 0:
                    @pl.when(t > 0)
                    def _():
                        stage(s + 1, 1)
                else:
                    @pl.when(t < (N_SUPERS // 2) - 1)
                    def _():
                        stage(s + 1, 0)

                for p in range(SUPER):
                    b = p % 2
                    wait(b)
                    scatter_add(par, p, b)
                    if p + 2 < SUPER:
                        start(par, p + 2, b)
                    else:
                        nxt = 1 - par
                        if half == 0:
                            if p == SUPER - 2:
                                wait_idx(nxt)
                            start(nxt, p + 2 - SUPER, b)
                        else:
                            @pl.when(t < (N_SUPERS // 2) - 1)
                            def _(p=p, nxt=nxt, b=b):
                                if p == SUPER - 2:
                                    wait_idx(nxt)
                                start(nxt, p + 2 - SUPER, b)
            return carry

        lax.fori_loop(0, N_SUPERS // 2, pair, 0)

        plsc.subcore_barrier()

        @pl.when(sid < 15)
        def _():
            pltpu.sync_copy(acc.at[pl.ds(sid * OUT_ROWS_MAIN, OUT_ROWS_MAIN)],
                            out.at[pl.ds(sid * OUT_ROWS_MAIN, OUT_ROWS_MAIN)])

        @pl.when(sid == 15)
        def _():
            pltpu.sync_copy(acc.at[pl.ds(15 * OUT_ROWS_MAIN, OUT_ROWS_LAST)],
                            out.at[pl.ds(15 * OUT_ROWS_MAIN, OUT_ROWS_LAST)])

    @pl.when(cid == 0)
    def _():
        run(h_hbm, hs_out)

    @pl.when(cid == 1)
    def _():
        run(prod_hbm, fc_out)


@functools.partial(
    pl.kernel,
    out_type=[
        jax.ShapeDtypeStruct((N_NODES, D), jnp.float32),
        jax.ShapeDtypeStruct((N_NODES, D), jnp.float32),
    ],
    mesh=plsc.VectorSubcoreMesh(core_axis_name="c", subcore_axis_name="s"),
    scratch_types=[
        pltpu.VMEM((SUPER, CHUNK), jnp.int32),
        pltpu.VMEM((SUPER, CHUNK), jnp.int32),
        pltpu.VMEM((SUPER, CHUNK), jnp.int32),
        pltpu.VMEM((SUPER, CHUNK), jnp.int32),
        pltpu.VMEM((CHUNK, D), jnp.float32),
        pltpu.VMEM((CHUNK, D), jnp.float32),
        pltpu.VMEM_SHARED((ACC_ROWS, D), jnp.float32),
        pltpu.SemaphoreType.DMA,
        pltpu.SemaphoreType.DMA,
        pltpu.SemaphoreType.DMA,
    ],
)
def _sc_segsum(h_hbm, prod_hbm, src_hbm, dst_hbm, zero_hbm, hs_out, fc_out,
               si0, si1, di0, di1, buf0, buf1, acc, sem0, sem1, isem):
    _sc_body(h_hbm, prod_hbm, src_hbm, dst_hbm, zero_hbm, hs_out, fc_out,
             si0, si1, di0, di1, buf0, buf1, acc, sem0, sem1, isem)


def _post_body(hs_ref, fc_ref, wg_ref, bg_ref, out_ref):
    gates = jnp.dot(hs_ref[...], wg_ref[...], preferred_element_type=jnp.float32)
    gates = gates + bg_ref[...]
    i = jax.nn.sigmoid(gates[:, :D])
    o = jax.nn.sigmoid(gates[:, D:2 * D])
    g = jnp.tanh(gates[:, 2 * D:])
    c_new = i * g + fc_ref[...]
    h_new = o * jnp.tanh(c_new)
    out_ref[:, 0, :] = h_new
    out_ref[:, 1, :] = c_new


def _post(hs_sum, fc_reduce, Wg, bg2d):
    return pl.pallas_call(
        _post_body,
        grid=(N_NODES // TC_BLOCK,),
        in_specs=[
            pl.BlockSpec((TC_BLOCK, D), lambda i: (i, 0)),
            pl.BlockSpec((TC_BLOCK, D), lambda i: (i, 0)),
            pl.BlockSpec((D, 3 * D), lambda i: (0, 0)),
            pl.BlockSpec((1, 3 * D), lambda i: (0, 0)),
        ],
        out_specs=pl.BlockSpec((TC_BLOCK, 2, D), lambda i: (i, 0, 0)),
        out_shape=jax.ShapeDtypeStruct((N_NODES, 2, D), jnp.float32),
    )(hs_sum, fc_reduce, Wg, bg2d)


def kernel(h, c, edge_index, Wg, bg, Wf, bf):
    src = edge_index[0]
    dst = edge_index[1]
    pad = E_PAD - N_EDGES
    src_p = jnp.concatenate([src, jnp.zeros((pad,), jnp.int32)])
    dst_p = jnp.concatenate([dst, jnp.full((pad,), N_NODES, jnp.int32)])
    src_p = src_p.reshape(N_SUBCORES, CHUNKS_PER_TILE, CHUNK).astype(jnp.int32)
    dst_p = dst_p.reshape(N_SUBCORES, CHUNKS_PER_TILE, CHUNK).astype(jnp.int32)
    zeros = jnp.zeros((ACC_ROWS, D), jnp.float32)

    prod = _pre(h, c, Wf, bf.reshape(1, D))
    hs_sum, fc_reduce = _sc_segsum(h, prod, src_p, dst_p, zeros)
    return _post(hs_sum, fc_reduce, Wg, bg.reshape(1, 3 * D))
